# baseline (speedup 1.0000x reference)
import functools

import jax
import jax.numpy as jnp
from jax import lax
from jax.experimental import pallas as pl
from jax.experimental.pallas import tpu as pltpu
from jax.experimental.pallas import tpu_sc as plsc

NCORES = 2
NSUB = 16
NW = NCORES * NSUB

G = 512
SEG = 100
NSTREAM = 2
C = SEG * NSTREAM


def _sc_pool(emb, batch2d, ones_hbm_a, zeros_hbm_a):
    N, D = emb.shape
    nchunks = N // C
    jmax = (nchunks + NW - 1) // NW
    mesh = plsc.VectorSubcoreMesh(core_axis_name="c", subcore_axis_name="s",
                                  num_cores=NCORES, num_subcores=NSUB)

    @functools.partial(
        pl.kernel,
        out_type=(
            jax.ShapeDtypeStruct((NCORES, G, D), jnp.float32),
            jax.ShapeDtypeStruct((NCORES, G, D), jnp.float32),
        ),
        mesh=mesh,
        scratch_types=[
            pltpu.VMEM((C, D), jnp.float32),
            pltpu.VMEM((C, D), jnp.float32),
            pltpu.VMEM((NSTREAM, SEG), jnp.int32),
            pltpu.VMEM((NSTREAM, SEG), jnp.int32),
            pltpu.VMEM((SEG, D), jnp.float32),
            pltpu.VMEM_SHARED((G, D), jnp.float32),
            pltpu.VMEM_SHARED((G, D), jnp.float32),
            pltpu.SemaphoreType.DMA,
            pltpu.SemaphoreType.DMA,
            pltpu.SemaphoreType.DMA,
        ],
    )
    def sc_kernel(emb_hbm, batch_hbm, ones_hbm, zeros_hbm,
                  pooled_out, counts_out,
                  rows0, rows1, idx0, idx1, ones_v, acc_sh, cnt_sh,
                  sem0, sem1, sem_s):
        cid = lax.axis_index("c")
        sid = lax.axis_index("s")
        wid = sid * NCORES + cid
        R = G // NSUB

        pltpu.sync_copy(ones_hbm, ones_v)
        pltpu.sync_copy(zeros_hbm.at[pl.ds(R * sid, R)],
                        acc_sh.at[pl.ds(R * sid, R)])
        pltpu.sync_copy(zeros_hbm.at[pl.ds(R * sid, R)],
                        cnt_sh.at[pl.ds(R * sid, R)])

        plsc.subcore_barrier()

        rows = (rows0, rows1)
        idxs = (idx0, idx1)
        sems = (sem0, sem1)

        def start(j):
            b = j % 2
            k = wid + NW * j

            @pl.when(k < nchunks)
            def _():
                pltpu.async_copy(emb_hbm.at[pl.ds(k * C, C)], rows[b],
                                 sems[b])
                pltpu.async_copy(batch_hbm.at[pl.ds(k * NSTREAM, NSTREAM)],
                                 idxs[b], sems[b])

        def process(j):
            b = j % 2
            k = wid + NW * j

            @pl.when(k < nchunks)
            def _():
                pltpu.make_async_copy(emb_hbm.at[pl.ds(k * C, C)], rows[b],
                                      sems[b]).wait()
                pltpu.make_async_copy(
                    batch_hbm.at[pl.ds(k * NSTREAM, NSTREAM)], idxs[b],
                    sems[b]).wait()
                for m in range(NSTREAM):
                    pltpu.sync_copy(rows[b].at[pl.ds(m * SEG, SEG)],
                                    acc_sh.at[idxs[b].at[m]], add=True)
                    pltpu.sync_copy(ones_v, cnt_sh.at[idxs[b].at[m]],
                                    add=True)

        start(0)
        for j in range(jmax):
            if j + 1 < jmax:
                start(j + 1)
            process(j)

        plsc.subcore_barrier()

        @pl.when(sid == 0)
        def _():
            pltpu.sync_copy(acc_sh, pooled_out.at[cid])
            pltpu.sync_copy(cnt_sh, counts_out.at[cid])

    return sc_kernel(emb, batch2d, ones_hbm_a, zeros_hbm_a)


def _head_body(pooled_ref, cnts_ref, w_ref, b_ref, out_ref):
    acc = pooled_ref[0] + pooled_ref[1]
    cnt = cnts_ref[:, 0:1] + cnts_ref[:, 1:2]
    c = jnp.maximum(cnt, 1.0)
    g = acc / c
    out_ref[...] = lax.dot_general(
        g, w_ref[...], (((1,), (1,)), ((), ())),
        preferred_element_type=jnp.float32) + b_ref[...]


def kernel(node_embeddings, batch, W, b):
    N, D = node_embeddings.shape
    K = W.shape[0]
    batch2d = batch.reshape(N // SEG, SEG)
    ones_a = jnp.ones((SEG, D), jnp.float32)
    zeros_a = jnp.zeros((G, D), jnp.float32)
    pooled, counts = _sc_pool(node_embeddings, batch2d, ones_a, zeros_a)
    cnt2 = counts[:, :, 0].T
    head = pl.pallas_call(
        _head_body,
        out_shape=jax.ShapeDtypeStruct((G, K), jnp.float32),
    )
    return head(pooled, cnt2, W, b.reshape(1, K))


# speedup vs baseline: 1.0549x; 1.0549x over previous
import functools

import jax
import jax.numpy as jnp
from jax import lax
from jax.experimental import pallas as pl
from jax.experimental.pallas import tpu as pltpu
from jax.experimental.pallas import tpu_sc as plsc

NCORES = 2
NSUB = 16
NW = NCORES * NSUB

G = 512
SEG = 80
NSTREAM = 2
C = SEG * NSTREAM


def _sc_pool(emb, batch2d, ones_hbm_a, zeros_hbm_a):
    N, D = emb.shape
    nchunks = N // C
    jmax = (nchunks + NW - 1) // NW
    mesh = plsc.VectorSubcoreMesh(core_axis_name="c", subcore_axis_name="s",
                                  num_cores=NCORES, num_subcores=NSUB)

    @functools.partial(
        pl.kernel,
        out_type=(
            jax.ShapeDtypeStruct((NCORES, G, D), jnp.float32),
            jax.ShapeDtypeStruct((NCORES, G, D), jnp.float32),
        ),
        mesh=mesh,
        scratch_types=[
            pltpu.VMEM((C, D), jnp.float32),
            pltpu.VMEM((C, D), jnp.float32),
            pltpu.VMEM((C, D), jnp.float32),
            pltpu.VMEM((NSTREAM, SEG), jnp.int32),
            pltpu.VMEM((NSTREAM, SEG), jnp.int32),
            pltpu.VMEM((NSTREAM, SEG), jnp.int32),
            pltpu.VMEM((SEG, D), jnp.float32),
            pltpu.VMEM_SHARED((G, D), jnp.float32),
            pltpu.VMEM_SHARED((G, D), jnp.float32),
            pltpu.SemaphoreType.DMA,
            pltpu.SemaphoreType.DMA,
            pltpu.SemaphoreType.DMA,
            pltpu.SemaphoreType.DMA,
        ],
    )
    def sc_kernel(emb_hbm, batch_hbm, ones_hbm, zeros_hbm,
                  pooled_out, counts_out,
                  rows0, rows1, rows2, idx0, idx1, idx2, ones_v,
                  acc_sh, cnt_sh, sem0, sem1, sem2, sem_s):
        cid = lax.axis_index("c")
        sid = lax.axis_index("s")
        wid = sid * NCORES + cid
        R = G // NSUB

        pltpu.sync_copy(ones_hbm, ones_v)
        pltpu.sync_copy(zeros_hbm.at[pl.ds(R * sid, R)],
                        acc_sh.at[pl.ds(R * sid, R)])
        pltpu.sync_copy(zeros_hbm.at[pl.ds(R * sid, R)],
                        cnt_sh.at[pl.ds(R * sid, R)])

        plsc.subcore_barrier()

        rows = (rows0, rows1, rows2)
        idxs = (idx0, idx1, idx2)
        sems = (sem0, sem1, sem2)

        def start(j):
            b = j % 3
            k = wid + NW * j

            @pl.when(k < nchunks)
            def _():
                pltpu.async_copy(emb_hbm.at[pl.ds(k * C, C)], rows[b],
                                 sems[b])
                pltpu.async_copy(batch_hbm.at[pl.ds(k * NSTREAM, NSTREAM)],
                                 idxs[b], sems[b])

        def process(j):
            b = j % 3
            k = wid + NW * j

            @pl.when(k < nchunks)
            def _():
                pltpu.make_async_copy(emb_hbm.at[pl.ds(k * C, C)], rows[b],
                                      sems[b]).wait()
                pltpu.make_async_copy(
                    batch_hbm.at[pl.ds(k * NSTREAM, NSTREAM)], idxs[b],
                    sems[b]).wait()
                for m in range(NSTREAM):
                    pltpu.sync_copy(rows[b].at[pl.ds(m * SEG, SEG)],
                                    acc_sh.at[idxs[b].at[m]], add=True)
                    pltpu.sync_copy(ones_v, cnt_sh.at[idxs[b].at[m]],
                                    add=True)

        start(0)
        start(1)
        for j in range(jmax):
            if j + 2 < jmax:
                start(j + 2)
            process(j)

        plsc.subcore_barrier()

        @pl.when(sid == 0)
        def _():
            pltpu.sync_copy(acc_sh, pooled_out.at[cid])
            pltpu.sync_copy(cnt_sh, counts_out.at[cid])

    return sc_kernel(emb, batch2d, ones_hbm_a, zeros_hbm_a)


def _head_body(pooled_ref, cnts_ref, w_ref, b_ref, out_ref):
    acc = pooled_ref[0] + pooled_ref[1]
    cnt = cnts_ref[:, 0:1] + cnts_ref[:, 1:2]
    c = jnp.maximum(cnt, 1.0)
    g = acc / c
    out_ref[...] = lax.dot_general(
        g, w_ref[...], (((1,), (1,)), ((), ())),
        preferred_element_type=jnp.float32) + b_ref[...]


def kernel(node_embeddings, batch, W, b):
    N, D = node_embeddings.shape
    K = W.shape[0]
    batch2d = batch.reshape(N // SEG, SEG)
    ones_a = jnp.ones((SEG, D), jnp.float32)
    zeros_a = jnp.zeros((G, D), jnp.float32)
    pooled, counts = _sc_pool(node_embeddings, batch2d, ones_a, zeros_a)
    cnt2 = counts[:, :, 0].T
    head = pl.pallas_call(
        _head_body,
        out_shape=jax.ShapeDtypeStruct((G, K), jnp.float32),
    )
    return head(pooled, cnt2, W, b.reshape(1, K))


# speedup vs baseline: 1.3341x; 1.2647x over previous
import functools

import jax
import jax.numpy as jnp
from jax import lax
from jax.experimental import pallas as pl
from jax.experimental.pallas import tpu as pltpu
from jax.experimental.pallas import tpu_sc as plsc

NCORES = 2
NSUB = 16
NW = NCORES * NSUB

G = 512
SEG = 80
NSTREAM = 2
C = SEG * NSTREAM


def _sc_pool(emb, batch2d, ones_hbm_a, zeros_hbm_a):
    N, D = emb.shape
    nchunks = N // C
    jmax = (nchunks + NW - 1) // NW
    mesh = plsc.VectorSubcoreMesh(core_axis_name="c", subcore_axis_name="s",
                                  num_cores=NCORES, num_subcores=NSUB)

    @functools.partial(
        pl.kernel,
        out_type=(
            jax.ShapeDtypeStruct((NCORES, G, D), jnp.float32),
            jax.ShapeDtypeStruct((NCORES, G, D), jnp.float32),
        ),
        mesh=mesh,
        scratch_types=[
            pltpu.VMEM((C, D), jnp.float32),
            pltpu.VMEM((C, D), jnp.float32),
            pltpu.VMEM((NSTREAM, SEG), jnp.int32),
            pltpu.VMEM((NSTREAM, SEG), jnp.int32),
            pltpu.VMEM((SEG, D), jnp.float32),
            pltpu.VMEM_SHARED((G, D), jnp.float32),
            pltpu.VMEM_SHARED((G, D), jnp.float32),
            pltpu.SemaphoreType.DMA,
            pltpu.SemaphoreType.DMA,
            pltpu.SemaphoreType.DMA,
        ],
    )
    def sc_kernel(emb_hbm, batch_hbm, ones_hbm, zeros_hbm,
                  pooled_out, counts_out,
                  rows0, rows1, idx0, idx1, ones_v, acc_sh, cnt_sh,
                  sem0, sem1, sem_s):
        cid = lax.axis_index("c")
        sid = lax.axis_index("s")
        wid = sid * NCORES + cid
        R = G // NSUB

        pltpu.sync_copy(ones_hbm, ones_v)
        pltpu.sync_copy(zeros_hbm.at[pl.ds(R * sid, R)],
                        acc_sh.at[pl.ds(R * sid, R)])
        pltpu.sync_copy(zeros_hbm.at[pl.ds(R * sid, R)],
                        cnt_sh.at[pl.ds(R * sid, R)])

        plsc.subcore_barrier()

        rows = (rows0, rows1)
        idxs = (idx0, idx1)
        sems = (sem0, sem1)

        def start(j):
            b = j % 2
            k = wid + NW * j

            @pl.when(k < nchunks)
            def _():
                pltpu.async_copy(emb_hbm.at[pl.ds(k * C, C)], rows[b],
                                 sems[b])
                pltpu.async_copy(batch_hbm.at[pl.ds(k * NSTREAM, NSTREAM)],
                                 idxs[b], sems[b])

        def process(j):
            b = j % 2
            k = wid + NW * j

            @pl.when(k < nchunks)
            def _():
                pltpu.make_async_copy(emb_hbm.at[pl.ds(k * C, C)], rows[b],
                                      sems[b]).wait()
                pltpu.make_async_copy(
                    batch_hbm.at[pl.ds(k * NSTREAM, NSTREAM)], idxs[b],
                    sems[b]).wait()
                for m in range(NSTREAM):
                    pltpu.sync_copy(rows[b].at[pl.ds(m * SEG, SEG)],
                                    acc_sh.at[idxs[b].at[m]], add=True)
                    pltpu.sync_copy(ones_v, cnt_sh.at[idxs[b].at[m]],
                                    add=True)

        start(0)
        for j in range(jmax):
            if j + 1 < jmax:
                start(j + 1)
            process(j)

        plsc.subcore_barrier()

        @pl.when(sid == 0)
        def _():
            pltpu.sync_copy(acc_sh, pooled_out.at[cid])
            pltpu.sync_copy(cnt_sh, counts_out.at[cid])

    return sc_kernel(emb, batch2d, ones_hbm_a, zeros_hbm_a)


def _head_body(pooled_ref, cnts_ref, w_ref, b_ref, out_ref):
    acc = pooled_ref[0] + pooled_ref[1]
    cnt = cnts_ref[0, :, 0:1] + cnts_ref[1, :, 0:1]
    c = jnp.maximum(cnt, 1.0)
    g = acc / c
    out_ref[...] = lax.dot_general(
        g, w_ref[...], (((1,), (1,)), ((), ())),
        preferred_element_type=jnp.float32) + b_ref[...]


def kernel(node_embeddings, batch, W, b):
    N, D = node_embeddings.shape
    K = W.shape[0]
    batch2d = batch.reshape(N // SEG, SEG)
    ones_a = jnp.ones((SEG, D), jnp.float32)
    zeros_a = jnp.zeros((G, D), jnp.float32)
    pooled, counts = _sc_pool(node_embeddings, batch2d, ones_a, zeros_a)
    head = pl.pallas_call(
        _head_body,
        out_shape=jax.ShapeDtypeStruct((G, K), jnp.float32),
    )
    return head(pooled, counts, W, b.reshape(1, K))
